# baseline (speedup 1.0000x reference)
import jax
import jax.numpy as jnp
from jax import lax
from jax.experimental import pallas as pl
from jax.experimental.pallas import tpu as pltpu
from jax.experimental.pallas import tpu_sc as plsc

NUM_HEADS = 16
HEAD_DIM = 16
HIDDEN = NUM_HEADS * HEAD_DIM
SCALE = float(HEAD_DIM) ** 0.5
NODES = 10000
EDGES = 160000

NC = 2
NS = 16
HH = NUM_HEADS // NC
HW = HH * HEAD_DIM
CHUNK = 32
N_CHUNKS = 321
E_PAD = NS * N_CHUNKS * CHUNK
WV_ROWS = 10016
ZB = WV_ROWS
ZDUMMY = ZB + NODES // 16
ACC_ROWS = 10688
GROUPS = CHUNK // 16


def _sc_body(kvtab, qtab, einfo, out,
             acc, kvbuf, qbuf, mz, eidx, scat,
             semi, semk, semq, semsc):
    c = lax.axis_index("c")
    s = lax.axis_index("s")
    zero16 = jnp.zeros((16,), jnp.float32)
    iota16 = lax.iota(jnp.int32, 16)

    @pl.loop(0, 2 * CHUNK)
    def _zero_rows(r):
        for cb in range(HW // 16):
            mz[0, r, pl.ds(cb * 16, 16)] = zero16
            mz[1, r, pl.ds(cb * 16, 16)] = zero16

    @pl.loop(0, 11)
    def _zero_acc(m):
        t = m * NS + s
        @pl.when(t < ACC_ROWS // (2 * CHUNK))
        def _():
            pltpu.sync_copy(mz.at[0], acc.at[pl.ds(t * 2 * CHUNK, 2 * CHUNK)])

    plsc.subcore_barrier()

    pltpu.sync_copy(einfo.at[c, s, 0], eidx.at[0, 0])
    for g in range(GROUPS):
        scat[1, 0, pl.ds(g * 16, 16)] = jnp.full((16,), NODES, jnp.int32)
        scat[1, 0, pl.ds(CHUNK + g * 16, 16)] = jnp.full((16,), ZDUMMY,
                                                         jnp.int32)
    pltpu.async_copy(mz.at[1], acc.at[scat.at[1, 0]], semsc, add=True)
    pltpu.async_copy(kvtab.at[eidx.at[0, 0, pl.ds(0, CHUNK)]], kvbuf.at[0],
                     semk)
    pltpu.async_copy(qtab.at[eidx.at[0, 0, pl.ds(CHUNK, CHUNK)]], qbuf.at[0],
                     semq)

    @pl.loop(0, N_CHUNKS)
    def _chunk(j):
        p = jnp.bitwise_and(j, 1)
        pn = 1 - p
        jn = jnp.minimum(j + 1, N_CHUNKS - 1)
        pv = jnp.full((16,), p, jnp.int32)

        pltpu.async_copy(einfo.at[c, s, jn], eidx.at[pn, 0], semi)

        for g in range(GROUPS):
            dv = eidx[p, 0, pl.ds(2 * CHUNK + g * 16, 16)]
            scat[p, 0, pl.ds(g * 16, 16)] = dv
            scat[p, 0, pl.ds(CHUNK + g * 16, 16)] = (
                ZB + lax.shift_right_logical(dv, 4))

        pltpu.make_async_copy(kvtab.at[eidx.at[p, 0, pl.ds(0, CHUNK)]],
                              kvbuf.at[p], semk).wait()
        pltpu.make_async_copy(qtab.at[eidx.at[p, 0, pl.ds(CHUNK, CHUNK)]],
                              qbuf.at[p], semq).wait()

        @pl.loop(0, GROUPS)
        def _score(g):
            rows = iota16 + g * 16
            zrows = rows + CHUNK
            dv = scat[p, 0, pl.ds(g * 16, 16)]
            zc0 = lax.shift_left(jnp.bitwise_and(dv, 15), 3)

            @pl.loop(0, HH)
            def _head(h):
                dot = zero16
                for d in range(HEAD_DIM):
                    col = h * HEAD_DIM + jnp.bitwise_and(d + iota16, 15)
                    kv = plsc.load_gather(kvbuf, [pv, rows, col])
                    qv = plsc.load_gather(qbuf, [pv, rows, col])
                    dot = dot + kv * qv
                sc = dot * (1.0 / SCALE)
                sc = jnp.minimum(jnp.maximum(sc, -5.0), 5.0)
                es = jnp.exp(sc)
                plsc.store_scatter(mz, [pv, zrows, zc0 + h], es)

        @pl.loop(0, GROUPS)
        def _scale(g):
            rows = iota16 + g * 16
            zrows = rows + CHUNK
            dv = scat[p, 0, pl.ds(g * 16, 16)]
            zc0 = lax.shift_left(jnp.bitwise_and(dv, 15), 3)

            @pl.loop(0, HH)
            def _head(h):
                es = plsc.load_gather(mz, [pv, zrows, zc0 + h])
                for d in range(HEAD_DIM):
                    col = h * HEAD_DIM + jnp.bitwise_and(d + iota16, 15)
                    vv = plsc.load_gather(kvbuf, [pv, rows, HW + col])
                    plsc.store_scatter(mz, [pv, rows, col], vv * es)

        pltpu.async_copy(mz.at[p], acc.at[scat.at[p, 0]], semsc, add=True)
        pltpu.make_async_copy(einfo.at[c, s, jn], eidx.at[pn, 0], semi).wait()
        pltpu.async_copy(kvtab.at[eidx.at[pn, 0, pl.ds(0, CHUNK)]],
                         kvbuf.at[pn], semk)
        pltpu.async_copy(qtab.at[eidx.at[pn, 0, pl.ds(CHUNK, CHUNK)]],
                         qbuf.at[pn], semq)

        pltpu.make_async_copy(mz.at[pn], acc.at[scat.at[pn, 0]], semsc).wait()

        pnv = jnp.full((16,), pn, jnp.int32)
        for g in range(GROUPS):
            rows = iota16 + g * 16
            zrows = rows + CHUNK
            dv = scat[pn, 0, pl.ds(g * 16, 16)]
            zc0 = lax.shift_left(jnp.bitwise_and(dv, 15), 3)
            for h in range(HH):
                plsc.store_scatter(mz, [pnv, zrows, zc0 + h], zero16)

    lastp = (N_CHUNKS - 1) % 2
    pltpu.make_async_copy(mz.at[lastp], acc.at[scat.at[lastp, 0]],
                          semsc).wait()
    pltpu.make_async_copy(kvtab.at[eidx.at[1 - lastp, 0, pl.ds(0, CHUNK)]],
                          kvbuf.at[1 - lastp], semk).wait()
    pltpu.make_async_copy(qtab.at[eidx.at[1 - lastp, 0, pl.ds(CHUNK, CHUNK)]],
                          qbuf.at[1 - lastp], semq).wait()

    plsc.subcore_barrier()

    @pl.loop(0, 20)
    def _norm(m):
        t = m * NS + s

        @pl.when(t < WV_ROWS // CHUNK)
        def _():
            base = t * CHUNK
            zoff = t * GROUPS
            zalign = jnp.bitwise_and(zoff, ~7)
            zdelta = zoff - zalign
            pltpu.sync_copy(acc.at[pl.ds(base, CHUNK)], qbuf.at[0])
            pltpu.sync_copy(acc.at[pl.ds(ZB + zalign, 16)],
                            qbuf.at[1, pl.ds(0, 16)])

            @pl.loop(0, CHUNK)
            def _node(n):
                zrow = jnp.full((16,),
                                zdelta + lax.shift_right_logical(n, 4),
                                jnp.int32)
                zc0 = lax.shift_left(jnp.bitwise_and(n, 15), 3)
                one = jnp.full((16,), 1, jnp.int32)
                for h in range(HH):
                    zcol = jnp.full((16,), zc0 + h, jnp.int32)
                    zh = plsc.load_gather(qbuf, [one, zrow, zcol])
                    wv = qbuf[0, n, pl.ds(h * HEAD_DIM, 16)]
                    mz[0, n, pl.ds(h * HEAD_DIM, 16)] = wv / (zh + 1e-6)

            pltpu.sync_copy(mz.at[0, pl.ds(0, CHUNK)],
                            out.at[c, pl.ds(base, CHUNK)])


@jax.jit
def _run(kvtab, qtab, einfo):
    mesh = plsc.VectorSubcoreMesh(core_axis_name="c", subcore_axis_name="s",
                                  num_cores=NC, num_subcores=NS)
    return pl.kernel(
        _sc_body,
        out_type=jax.ShapeDtypeStruct((NC, WV_ROWS, HW), jnp.float32),
        mesh=mesh,
        compiler_params=pltpu.CompilerParams(needs_layout_passes=False),
        scratch_types=[
            pltpu.VMEM_SHARED((ACC_ROWS, HW), jnp.float32),
            pltpu.VMEM((2, CHUNK, 2 * HW), jnp.float32),
            pltpu.VMEM((2, CHUNK, HW), jnp.float32),
            pltpu.VMEM((2, 2 * CHUNK, HW), jnp.float32),
            pltpu.VMEM((2, 1, 3 * CHUNK), jnp.int32),
            pltpu.VMEM((2, 1, 2 * CHUNK), jnp.int32),
            pltpu.SemaphoreType.DMA,
            pltpu.SemaphoreType.DMA,
            pltpu.SemaphoreType.DMA,
            pltpu.SemaphoreType.DMA,
        ],
    )(kvtab, qtab, einfo)


def kernel(q, k, v, edge_index):
    batch, node_num = q.shape[0], q.shape[1]

    def half_tab(x):
        return (x.reshape(NODES, NC, HW)
                 .transpose(1, 0, 2)
                 .reshape(NC * NODES, HW))

    ktab = half_tab(k)
    qtab = half_tab(q)
    vtab = half_tab(v)
    kvtab = jnp.concatenate([ktab, vtab], axis=1)

    src = edge_index[0].astype(jnp.int32)
    dst = edge_index[1].astype(jnp.int32)
    pad = E_PAD - EDGES
    src_p = jnp.concatenate([src, jnp.zeros((pad,), jnp.int32)])
    dst_gp = jnp.concatenate([dst, jnp.zeros((pad,), jnp.int32)])
    dst_sp = jnp.concatenate([dst, jnp.full((pad,), NODES, jnp.int32)])
    srcr = src_p.reshape(NS, N_CHUNKS, CHUNK)
    dstr = dst_gp.reshape(NS, N_CHUNKS, CHUNK)
    dssr = dst_sp.reshape(NS, N_CHUNKS, CHUNK)
    einfo = jnp.stack([
        jnp.concatenate([srcr + cc * NODES, dstr + cc * NODES, dssr], axis=-1)
        for cc in range(NC)])

    out2 = _run(kvtab, qtab, einfo)
    return out2[:, :NODES].transpose(1, 0, 2).reshape(batch, node_num, HIDDEN)
